# baseline (device time: 53838 ns/iter reference)
import jax
import jax.numpy as jnp
from jax import lax
from jax.experimental import pallas as pl
from jax.experimental.pallas import tpu as pltpu

D_OUT = 1024
F = 4096
HALF = D_OUT // 2
Q = HALF // 2
CHUNKS = ((0, 512), (512, 1536), (2048, 1536), (3584, 512))
NC = len(CHUNKS)

_CONTRACT0 = (((0,), (0,)), ((), ()))


def kernel(x, dy):
    m_per, d = x.shape
    _, f = dy.shape
    assert d == D_OUT and f == F

    def body(x_ref, dy_ref, out_ref, *scratch):
        sendx = scratch[0:NC]
        xq = scratch[NC : 2 * NC]
        yq = scratch[2 * NC : 3 * NC]
        xs_sems, xr_sems, ys_sems, yr_sems = scratch[3 * NC :]
        mx = lax.axis_index("x")
        my = lax.axis_index("y")
        mz = lax.axis_index("z")
        px = 1 - mx
        py = 1 - my
        xpeer = (px, my, mz)
        ypeer = (mx, py, mz)

        barrier_sem = pltpu.get_barrier_semaphore()
        for nbr in (xpeer, ypeer):
            pl.semaphore_signal(
                barrier_sem, inc=1,
                device_id=nbr, device_id_type=pl.DeviceIdType.MESH,
            )
        pl.semaphore_wait(barrier_sem, 2)

        qsend_off = px * HALF + my * Q
        x_rdmas = []
        for c, (off, fc) in enumerate(CHUNKS):
            sendx[c][...] = lax.dot_general(
                x_ref[:, pl.ds(qsend_off, Q)],
                dy_ref[:, off : off + fc],
                dimension_numbers=_CONTRACT0,
                preferred_element_type=jnp.float32,
            ).astype(jnp.bfloat16)
            r = pltpu.make_async_remote_copy(
                src_ref=sendx[c],
                dst_ref=xq[c],
                send_sem=xs_sems.at[c],
                recv_sem=xr_sems.at[c],
                device_id=xpeer,
                device_id_type=pl.DeviceIdType.MESH,
            )
            r.start()
            x_rdmas.append(r)

        def _accum(row_block, buf_ref, off, fc):
            if row_block == 0:
                out_ref[0:Q, off : off + fc] += buf_ref[...].astype(
                    jnp.float32
                )
            else:
                out_ref[Q : 2 * Q, off : off + fc] += buf_ref[...].astype(
                    jnp.float32
                )

        y_rdmas = []
        for c, (off, fc) in enumerate(CHUNKS):
            x_rdmas[c].wait_recv()
            r = pltpu.make_async_remote_copy(
                src_ref=xq[c],
                dst_ref=yq[c],
                send_sem=ys_sems.at[c],
                recv_sem=yr_sems.at[c],
                device_id=ypeer,
                device_id_type=pl.DeviceIdType.MESH,
            )
            r.start()
            y_rdmas.append(r)
            out_ref[:, off : off + fc] = lax.dot_general(
                x_ref[:, pl.ds(mx * HALF, HALF)],
                dy_ref[:, off : off + fc],
                dimension_numbers=_CONTRACT0,
                preferred_element_type=jnp.float32,
            )
            pl.when(my == 0)(lambda b=xq[c], o=off, f_=fc: _accum(0, b, o, f_))
            pl.when(my == 1)(lambda b=xq[c], o=off, f_=fc: _accum(1, b, o, f_))

        for c, (off, fc) in enumerate(CHUNKS):
            y_rdmas[c].wait_recv()
            pl.when(my == 0)(lambda b=yq[c], o=off, f_=fc: _accum(1, b, o, f_))
            pl.when(my == 1)(lambda b=yq[c], o=off, f_=fc: _accum(0, b, o, f_))

        for c in range(NC):
            x_rdmas[c].wait_send()
            y_rdmas[c].wait_send()

    return pl.pallas_call(
        body,
        out_shape=jax.ShapeDtypeStruct((HALF, F), jnp.float32),
        in_specs=[
            pl.BlockSpec(memory_space=pltpu.VMEM),
            pl.BlockSpec(memory_space=pltpu.VMEM),
        ],
        out_specs=pl.BlockSpec(memory_space=pltpu.VMEM),
        scratch_shapes=(
            [pltpu.VMEM((Q, fc), jnp.bfloat16) for _, fc in CHUNKS]
            + [pltpu.VMEM((Q, fc), jnp.bfloat16) for _, fc in CHUNKS]
            + [pltpu.VMEM((Q, fc), jnp.bfloat16) for _, fc in CHUNKS]
            + [
                pltpu.SemaphoreType.DMA((NC,)),
                pltpu.SemaphoreType.DMA((NC,)),
                pltpu.SemaphoreType.DMA((NC,)),
                pltpu.SemaphoreType.DMA((NC,)),
            ]
        ),
        compiler_params=pltpu.CompilerParams(
            collective_id=0,
            vmem_limit_bytes=100 * 1024 * 1024,
        ),
    )(x, dy)


# device time: 51352 ns/iter; 1.0484x vs baseline; 1.0484x over previous
import jax
import jax.numpy as jnp
from jax import lax
from jax.experimental import pallas as pl
from jax.experimental.pallas import tpu as pltpu

D_OUT = 1024
F = 4096
HALF = D_OUT // 2
Q = HALF // 2
CHUNKS = ((0, 1024), (1024, 1024), (2048, 1024), (3072, 1024))
NC = len(CHUNKS)

_CONTRACT0 = (((0,), (0,)), ((), ()))


def kernel(x, dy):
    m_per, d = x.shape
    _, f = dy.shape
    assert d == D_OUT and f == F

    def body(x_ref, dy_ref, out_ref, *scratch):
        sendx = scratch[0:NC]
        xq = scratch[NC : 2 * NC]
        yq = scratch[2 * NC : 3 * NC]
        xs_sems, xr_sems, ys_sems, yr_sems = scratch[3 * NC :]
        mx = lax.axis_index("x")
        my = lax.axis_index("y")
        mz = lax.axis_index("z")
        px = 1 - mx
        py = 1 - my
        xpeer = (px, my, mz)
        ypeer = (mx, py, mz)

        barrier_sem = pltpu.get_barrier_semaphore()
        for nbr in (xpeer, ypeer):
            pl.semaphore_signal(
                barrier_sem, inc=1,
                device_id=nbr, device_id_type=pl.DeviceIdType.MESH,
            )
        pl.semaphore_wait(barrier_sem, 2)

        qsend_off = px * HALF + my * Q
        x_rdmas = []
        for c, (off, fc) in enumerate(CHUNKS):
            sendx[c][...] = lax.dot_general(
                x_ref[:, pl.ds(qsend_off, Q)].astype(jnp.bfloat16),
                dy_ref[:, off : off + fc].astype(jnp.bfloat16),
                dimension_numbers=_CONTRACT0,
                preferred_element_type=jnp.float32,
            ).astype(jnp.bfloat16)
            r = pltpu.make_async_remote_copy(
                src_ref=sendx[c],
                dst_ref=xq[c],
                send_sem=xs_sems.at[c],
                recv_sem=xr_sems.at[c],
                device_id=xpeer,
                device_id_type=pl.DeviceIdType.MESH,
            )
            r.start()
            x_rdmas.append(r)

        def _accum(row_block, buf_ref, off, fc):
            if row_block == 0:
                out_ref[0:Q, off : off + fc] += buf_ref[...].astype(
                    jnp.float32
                )
            else:
                out_ref[Q : 2 * Q, off : off + fc] += buf_ref[...].astype(
                    jnp.float32
                )

        y_rdmas = []
        for c, (off, fc) in enumerate(CHUNKS):
            x_rdmas[c].wait_recv()
            r = pltpu.make_async_remote_copy(
                src_ref=xq[c],
                dst_ref=yq[c],
                send_sem=ys_sems.at[c],
                recv_sem=yr_sems.at[c],
                device_id=ypeer,
                device_id_type=pl.DeviceIdType.MESH,
            )
            r.start()
            y_rdmas.append(r)
            out_ref[:, off : off + fc] = lax.dot_general(
                x_ref[:, pl.ds(mx * HALF, HALF)],
                dy_ref[:, off : off + fc],
                dimension_numbers=_CONTRACT0,
                preferred_element_type=jnp.float32,
            )
            pl.when(my == 0)(lambda b=xq[c], o=off, f_=fc: _accum(0, b, o, f_))
            pl.when(my == 1)(lambda b=xq[c], o=off, f_=fc: _accum(1, b, o, f_))

        for c, (off, fc) in enumerate(CHUNKS):
            y_rdmas[c].wait_recv()
            pl.when(my == 0)(lambda b=yq[c], o=off, f_=fc: _accum(1, b, o, f_))
            pl.when(my == 1)(lambda b=yq[c], o=off, f_=fc: _accum(0, b, o, f_))

        for c in range(NC):
            x_rdmas[c].wait_send()
            y_rdmas[c].wait_send()

    return pl.pallas_call(
        body,
        out_shape=jax.ShapeDtypeStruct((HALF, F), jnp.float32),
        in_specs=[
            pl.BlockSpec(memory_space=pltpu.VMEM),
            pl.BlockSpec(memory_space=pltpu.VMEM),
        ],
        out_specs=pl.BlockSpec(memory_space=pltpu.VMEM),
        scratch_shapes=(
            [pltpu.VMEM((Q, fc), jnp.bfloat16) for _, fc in CHUNKS]
            + [pltpu.VMEM((Q, fc), jnp.bfloat16) for _, fc in CHUNKS]
            + [pltpu.VMEM((Q, fc), jnp.bfloat16) for _, fc in CHUNKS]
            + [
                pltpu.SemaphoreType.DMA((NC,)),
                pltpu.SemaphoreType.DMA((NC,)),
                pltpu.SemaphoreType.DMA((NC,)),
                pltpu.SemaphoreType.DMA((NC,)),
            ]
        ),
        compiler_params=pltpu.CompilerParams(
            collective_id=0,
            vmem_limit_bytes=100 * 1024 * 1024,
        ),
    )(x, dy)


# device time: 51122 ns/iter; 1.0531x vs baseline; 1.0045x over previous
import jax
import jax.numpy as jnp
from jax import lax
from jax.experimental import pallas as pl
from jax.experimental.pallas import tpu as pltpu

D_OUT = 1024
F = 4096
HALF = D_OUT // 2
Q = HALF // 2
CHUNKS = ((0, 1024), (1024, 1024), (2048, 1024), (3072, 1024))
NC = len(CHUNKS)

_CONTRACT0 = (((0,), (0,)), ((), ()))


def kernel(x, dy):
    m_per, d = x.shape
    _, f = dy.shape
    assert d == D_OUT and f == F

    def body(x_ref, dy_ref, out_ref, *scratch):
        sendx = scratch[0:NC]
        xq = scratch[NC : 2 * NC]
        yq = scratch[2 * NC : 3 * NC]
        xs_sems, xr_sems, ys_sems, yr_sems = scratch[3 * NC :]
        mx = lax.axis_index("x")
        my = lax.axis_index("y")
        mz = lax.axis_index("z")
        px = 1 - mx
        py = 1 - my
        xpeer = (px, my, mz)
        ypeer = (mx, py, mz)

        barrier_sem = pltpu.get_barrier_semaphore()
        for nbr in (xpeer, ypeer):
            pl.semaphore_signal(
                barrier_sem, inc=1,
                device_id=nbr, device_id_type=pl.DeviceIdType.MESH,
            )

        qsend_off = px * HALF + my * Q
        x_rdmas = []
        for c, (off, fc) in enumerate(CHUNKS):
            sendx[c][...] = lax.dot_general(
                x_ref[:, pl.ds(qsend_off, Q)].astype(jnp.bfloat16),
                dy_ref[:, off : off + fc].astype(jnp.bfloat16),
                dimension_numbers=_CONTRACT0,
                preferred_element_type=jnp.float32,
            ).astype(jnp.bfloat16)
            if c == 0:
                pl.semaphore_wait(barrier_sem, 2)
            r = pltpu.make_async_remote_copy(
                src_ref=sendx[c],
                dst_ref=xq[c],
                send_sem=xs_sems.at[c],
                recv_sem=xr_sems.at[c],
                device_id=xpeer,
                device_id_type=pl.DeviceIdType.MESH,
            )
            r.start()
            x_rdmas.append(r)

        def _accum(row_block, buf_ref, off, fc):
            if row_block == 0:
                out_ref[0:Q, off : off + fc] += buf_ref[...].astype(
                    jnp.float32
                )
            else:
                out_ref[Q : 2 * Q, off : off + fc] += buf_ref[...].astype(
                    jnp.float32
                )

        y_rdmas = []
        for c, (off, fc) in enumerate(CHUNKS):
            x_rdmas[c].wait_recv()
            r = pltpu.make_async_remote_copy(
                src_ref=xq[c],
                dst_ref=yq[c],
                send_sem=ys_sems.at[c],
                recv_sem=yr_sems.at[c],
                device_id=ypeer,
                device_id_type=pl.DeviceIdType.MESH,
            )
            r.start()
            y_rdmas.append(r)
            out_ref[:, off : off + fc] = lax.dot_general(
                x_ref[:, pl.ds(mx * HALF, HALF)],
                dy_ref[:, off : off + fc],
                dimension_numbers=_CONTRACT0,
                preferred_element_type=jnp.float32,
            )
            pl.when(my == 0)(lambda b=xq[c], o=off, f_=fc: _accum(0, b, o, f_))
            pl.when(my == 1)(lambda b=xq[c], o=off, f_=fc: _accum(1, b, o, f_))

        for c, (off, fc) in enumerate(CHUNKS):
            y_rdmas[c].wait_recv()
            pl.when(my == 0)(lambda b=yq[c], o=off, f_=fc: _accum(1, b, o, f_))
            pl.when(my == 1)(lambda b=yq[c], o=off, f_=fc: _accum(0, b, o, f_))

        for c in range(NC):
            x_rdmas[c].wait_send()
            y_rdmas[c].wait_send()

    return pl.pallas_call(
        body,
        out_shape=jax.ShapeDtypeStruct((HALF, F), jnp.float32),
        in_specs=[
            pl.BlockSpec(memory_space=pltpu.VMEM),
            pl.BlockSpec(memory_space=pltpu.VMEM),
        ],
        out_specs=pl.BlockSpec(memory_space=pltpu.VMEM),
        scratch_shapes=(
            [pltpu.VMEM((Q, fc), jnp.bfloat16) for _, fc in CHUNKS]
            + [pltpu.VMEM((Q, fc), jnp.bfloat16) for _, fc in CHUNKS]
            + [pltpu.VMEM((Q, fc), jnp.bfloat16) for _, fc in CHUNKS]
            + [
                pltpu.SemaphoreType.DMA((NC,)),
                pltpu.SemaphoreType.DMA((NC,)),
                pltpu.SemaphoreType.DMA((NC,)),
                pltpu.SemaphoreType.DMA((NC,)),
            ]
        ),
        compiler_params=pltpu.CompilerParams(
            collective_id=0,
            vmem_limit_bytes=100 * 1024 * 1024,
        ),
    )(x, dy)
